# baseline (speedup 1.0000x reference)
import functools

import jax
import jax.numpy as jnp
from jax import lax
from jax.experimental import pallas as pl
from jax.experimental.pallas import tpu as pltpu
from jax.experimental.pallas import tpu_sc as plsc

BATCH = 16384
EMB = 32
HID = 64
NROWS = 1000000
_RPS = 4
_SLAB = _RPS * EMB

_TBLK = 2048
_TGRID = -(-NROWS // _TBLK)
_PACKED = 128 * (-(-NROWS // (128 * _RPS)))

_NC = 2
_NS = 16
_NW = _NC * _NS
_BPW = BATCH // _NW
_CHUNK = 128
_NCHUNK = _BPW // _CHUNK
_L = 16
_NGRP = _BPW // _L
_PPW = _BPW // _RPS


def _pack_body(xT_ref, eye_ref, out_ref):
    x = xT_ref[...]
    for c in range(_RPS):
        xc = jnp.concatenate(
            [x[:, (4 * t + c) * 128:(4 * t + c + 1) * 128] for t in range(4)],
            axis=1)
        yc = lax.dot_general(xc, eye_ref[...], (((0,), (0,)), ((), ())),
                             preferred_element_type=jnp.float32)
        out_ref[:, c * EMB:(c + 1) * EMB] = yc


def _pack_call(tabT, eye32):
    return pl.pallas_call(
        _pack_body,
        grid=(_TGRID,),
        in_specs=[pl.BlockSpec((EMB, _TBLK), lambda g: (0, g)),
                  pl.BlockSpec((EMB, EMB), lambda g: (0, 0))],
        out_specs=pl.BlockSpec((_TBLK // _RPS, _SLAB), lambda g: (g, 0)),
        out_shape=jax.ShapeDtypeStruct((_PACKED, _SLAB), jnp.float32),
    )(tabT, eye32)


def _gather_one_table(idx2_h, tab_h, out_h, wid, idx2_v, slabidx_v, slab_v,
                      pack_v, sem):
    pltpu.sync_copy(idx2_h.at[pl.ds(wid * _NCHUNK, _NCHUNK)], idx2_v)
    for j in range(_NCHUNK):
        for l in range(_CHUNK // _L):
            v = idx2_v[j, pl.ds(l * _L, _L)]
            s_row = jnp.bitwise_or(
                lax.shift_left(lax.shift_right_logical(v, 9), 7),
                jnp.bitwise_and(v, 127))
            slabidx_v[j, pl.ds(l * _L, _L)] = s_row
    copies = [
        pltpu.async_copy(tab_h.at[slabidx_v.at[j]],
                         slab_v.at[pl.ds(j * _CHUNK, _CHUNK)], sem)
        for j in range(_NCHUNK)
    ]
    for c in copies:
        c.wait()

    def grp_body(g, _):
        i_vec = lax.iota(jnp.int32, _L) + g * _L
        iv = plsc.load_gather(
            idx2_v, [lax.shift_right_logical(i_vec, 7),
                     jnp.bitwise_and(i_vec, 127)])
        src_col0 = jnp.bitwise_and(lax.shift_right_logical(iv, 7), 3) * EMB
        dst_row = lax.shift_right_logical(i_vec, 2)
        dst_col0 = jnp.bitwise_and(i_vec, 3) * EMB
        for k in range(EMB):
            vals = plsc.load_gather(slab_v, [i_vec, src_col0 + k])
            plsc.store_scatter(pack_v, [dst_row, dst_col0 + k], vals)
        return 0

    lax.fori_loop(0, _NGRP, grp_body, 0)
    pltpu.sync_copy(pack_v, out_h.at[pl.ds(wid * _PPW, _PPW)])


def _gather_body(sidx2_h, ridx2_h, stab_h, rtab_h, sout_h, rout_h,
                 idx2_v, slabidx_v, slab_v, pack_v, sem):
    wid = lax.axis_index("s") * _NC + lax.axis_index("c")
    _gather_one_table(sidx2_h, stab_h, sout_h, wid, idx2_v, slabidx_v,
                      slab_v, pack_v, sem)
    _gather_one_table(ridx2_h, rtab_h, rout_h, wid, idx2_v, slabidx_v,
                      slab_v, pack_v, sem)


_gather_call = functools.partial(
    pl.kernel,
    out_type=[jax.ShapeDtypeStruct((BATCH // _RPS, _SLAB), jnp.float32),
              jax.ShapeDtypeStruct((BATCH // _RPS, _SLAB), jnp.float32)],
    mesh=plsc.VectorSubcoreMesh(core_axis_name="c", subcore_axis_name="s"),
    scratch_types=[pltpu.VMEM((_NCHUNK, _CHUNK), jnp.int32),
                   pltpu.VMEM((_NCHUNK, _CHUNK), jnp.int32),
                   pltpu.VMEM((_BPW, _SLAB), jnp.float32),
                   pltpu.VMEM((_PPW, _SLAB), jnp.float32),
                   pltpu.SemaphoreType.DMA],
    compiler_params=pltpu.CompilerParams(needs_layout_passes=False),
)(_gather_body)


_BLK = 256
_H4 = _RPS * HID


def _mlp_body(s_ref, r_ref, w1s_ref, w1r_ref, b1_ref, m_ref, b2_ref, out_ref):
    h = jnp.tanh(
        jnp.dot(s_ref[...], w1s_ref[...], preferred_element_type=jnp.float32)
        + jnp.dot(r_ref[...], w1r_ref[...], preferred_element_type=jnp.float32)
        + b1_ref[...])
    logit = jnp.dot(h, m_ref[...], preferred_element_type=jnp.float32)
    out_ref[...] = jax.nn.sigmoid(logit + b2_ref[0, 0])


def _mlp_call(s4, r4, w1s_blk, w1r_blk, b1t, m, b2):
    grid = (BATCH // _RPS) // _BLK
    return pl.pallas_call(
        _mlp_body,
        grid=(grid,),
        in_specs=[
            pl.BlockSpec((_BLK, _SLAB), lambda i: (i, 0)),
            pl.BlockSpec((_BLK, _SLAB), lambda i: (i, 0)),
            pl.BlockSpec((_SLAB, _H4), lambda i: (0, 0)),
            pl.BlockSpec((_SLAB, _H4), lambda i: (0, 0)),
            pl.BlockSpec((1, _H4), lambda i: (0, 0)),
            pl.BlockSpec((_H4, _RPS), lambda i: (0, 0)),
            pl.BlockSpec((1, 1), lambda i: (0, 0)),
        ],
        out_specs=pl.BlockSpec((_BLK, _RPS), lambda i: (i, 0)),
        out_shape=jax.ShapeDtypeStruct((BATCH // _RPS, _RPS), jnp.float32),
    )(s4, r4, w1s_blk, w1r_blk, b1t, m, b2)


def kernel(sender_idx_batch, receiver_idx_batch, sender_table, receiver_table,
           W1, b1, W2, b2):
    sidx = sender_idx_batch.astype(jnp.int32).reshape(_NW * _NCHUNK, _CHUNK)
    ridx = receiver_idx_batch.astype(jnp.int32).reshape(_NW * _NCHUNK, _CHUNK)
    eye32 = jnp.eye(EMB, dtype=jnp.float32)
    stab = _pack_call(sender_table.T, eye32)
    rtab = _pack_call(receiver_table.T, eye32)
    s4, r4 = _gather_call(sidx, ridx, stab, rtab)
    eye4 = jnp.eye(_RPS, dtype=jnp.float32)
    w1s_blk = jnp.kron(eye4, W1[:, :EMB].T)
    w1r_blk = jnp.kron(eye4, W1[:, EMB:].T)
    b1t = jnp.tile(b1, _RPS).reshape(1, _H4)
    m = jnp.kron(eye4, W2.T)
    b2r = b2.reshape(1, 1)
    out4 = _mlp_call(s4, r4, w1s_blk, w1r_blk, b1t, m, b2r)
    return out4.reshape(BATCH)


# speedup vs baseline: 1.0531x; 1.0531x over previous
import functools

import jax
import jax.numpy as jnp
from jax import lax
from jax.experimental import pallas as pl
from jax.experimental.pallas import tpu as pltpu
from jax.experimental.pallas import tpu_sc as plsc

BATCH = 16384
EMB = 32
HID = 64

_NC = 2
_NS = 16
_NW = _NC * _NS
_BPW = BATCH // _NW
_LAG = 32


def _gather_body(idx_h, tab_h, out_h, idx_v, sem):
    wid = lax.axis_index("s") * _NC + lax.axis_index("c")
    base = wid * _BPW
    pltpu.sync_copy(idx_h.at[pl.ds(base, _BPW)], idx_v)

    def wait_one():
        pltpu.make_async_copy(
            tab_h.at[pl.ds(0, 1)], out_h.at[pl.ds(0, 1)], sem).wait()

    def body(g, _):
        vec = idx_v[pl.ds(g * 16, 16)]
        for l in range(16):
            pltpu.async_copy(tab_h.at[pl.ds(vec[l], 1)],
                             out_h.at[pl.ds(base + g * 16 + l, 1)], sem)

        @pl.when(g >= _LAG // 16)
        def _():
            for _i in range(16):
                wait_one()
        return 0

    lax.fori_loop(0, _BPW // 16, body, 0)
    for _ in range(_LAG):
        wait_one()


_gather_call = functools.partial(
    pl.kernel,
    out_type=jax.ShapeDtypeStruct((BATCH, EMB), jnp.float32),
    mesh=plsc.VectorSubcoreMesh(core_axis_name="c", subcore_axis_name="s"),
    scratch_types=[pltpu.VMEM((_BPW,), jnp.int32),
                   pltpu.SemaphoreType.DMA],
    compiler_params=pltpu.CompilerParams(needs_layout_passes=False),
)(_gather_body)


_BLK = 1024


def _mlp_body(s_ref, r_ref, w1s_ref, w1r_ref, b1_ref, w2_ref, b2_ref, out_ref):
    h = jnp.tanh(
        jnp.dot(s_ref[...], w1s_ref[...], preferred_element_type=jnp.float32)
        + jnp.dot(r_ref[...], w1r_ref[...], preferred_element_type=jnp.float32)
        + b1_ref[...])
    logit = jnp.sum(h * w2_ref[...], axis=1) + b2_ref[0, 0]
    out_ref[...] = jax.nn.sigmoid(logit)


def _mlp_call(s_emb, r_emb, w1s, w1r, b1, w2, b2):
    grid = BATCH // _BLK
    return pl.pallas_call(
        _mlp_body,
        grid=(grid,),
        in_specs=[
            pl.BlockSpec((_BLK, EMB), lambda i: (i, 0)),
            pl.BlockSpec((_BLK, EMB), lambda i: (i, 0)),
            pl.BlockSpec((EMB, HID), lambda i: (0, 0)),
            pl.BlockSpec((EMB, HID), lambda i: (0, 0)),
            pl.BlockSpec((1, HID), lambda i: (0, 0)),
            pl.BlockSpec((1, HID), lambda i: (0, 0)),
            pl.BlockSpec((1, 1), lambda i: (0, 0)),
        ],
        out_specs=pl.BlockSpec((_BLK,), lambda i: (i,)),
        out_shape=jax.ShapeDtypeStruct((BATCH,), jnp.float32),
    )(s_emb, r_emb, w1s, w1r, b1, w2, b2)


def kernel(sender_idx_batch, receiver_idx_batch, sender_table, receiver_table,
           W1, b1, W2, b2):
    sidx = sender_idx_batch.astype(jnp.int32)
    ridx = receiver_idx_batch.astype(jnp.int32)
    s_emb = _gather_call(sidx, sender_table)
    r_emb = _gather_call(ridx, receiver_table)
    w1s = W1[:, :EMB].T
    w1r = W1[:, EMB:].T
    b1r = b1.reshape(1, HID)
    w2r = W2.reshape(1, HID)
    b2r = b2.reshape(1, 1)
    return _mlp_call(s_emb, r_emb, w1s, w1r, b1r, w2r, b2r)


# speedup vs baseline: 1.4450x; 1.3722x over previous
import functools

import jax
import jax.numpy as jnp
from jax import lax
from jax.experimental import pallas as pl
from jax.experimental.pallas import tpu as pltpu
from jax.experimental.pallas import tpu_sc as plsc

BATCH = 16384
EMB = 32
HID = 64

_NC = 2
_NS = 16
_NW = _NC * _NS
_BPW = BATCH // _NW
_LAG = 32


def _gather_one_table(idx_h, tab_h, out_h, wid, idx_v, rows_v, sem):
    base = wid * _BPW
    pltpu.sync_copy(idx_h.at[pl.ds(base, _BPW)], idx_v)

    def wait_one():
        pltpu.make_async_copy(
            tab_h.at[pl.ds(0, 1)], rows_v.at[pl.ds(0, 1)], sem).wait()

    def body(g, _):
        vec = idx_v[pl.ds(g * 16, 16)]
        for l in range(16):
            pltpu.async_copy(tab_h.at[pl.ds(vec[l], 1)],
                             rows_v.at[pl.ds(g * 16 + l, 1)], sem)

        @pl.when(g >= _LAG // 16)
        def _():
            for _i in range(16):
                wait_one()
        return 0

    lax.fori_loop(0, _BPW // 16, body, 0)
    for _ in range(_LAG):
        wait_one()
    pltpu.sync_copy(rows_v, out_h.at[pl.ds(base, _BPW)])


def _gather_body(sidx_h, ridx_h, stab_h, rtab_h, sout_h, rout_h,
                 idx_v, rows_v, sem):
    wid = lax.axis_index("s") * _NC + lax.axis_index("c")
    _gather_one_table(sidx_h, stab_h, sout_h, wid, idx_v, rows_v, sem)
    _gather_one_table(ridx_h, rtab_h, rout_h, wid, idx_v, rows_v, sem)


_gather_call = functools.partial(
    pl.kernel,
    out_type=[jax.ShapeDtypeStruct((BATCH, EMB), jnp.float32),
              jax.ShapeDtypeStruct((BATCH, EMB), jnp.float32)],
    mesh=plsc.VectorSubcoreMesh(core_axis_name="c", subcore_axis_name="s"),
    scratch_types=[pltpu.VMEM((_BPW,), jnp.int32),
                   pltpu.VMEM((_BPW, EMB), jnp.float32),
                   pltpu.SemaphoreType.DMA],
    compiler_params=pltpu.CompilerParams(needs_layout_passes=False),
)(_gather_body)


_BLK = 1024


def _mlp_body(s_ref, r_ref, w1s_ref, w1r_ref, b1_ref, w2_ref, b2_ref, out_ref):
    h = jnp.tanh(
        jnp.dot(s_ref[...], w1s_ref[...], preferred_element_type=jnp.float32)
        + jnp.dot(r_ref[...], w1r_ref[...], preferred_element_type=jnp.float32)
        + b1_ref[...])
    logit = jnp.sum(h * w2_ref[...], axis=1) + b2_ref[0, 0]
    out_ref[...] = jax.nn.sigmoid(logit)


def _mlp_call(s_emb, r_emb, w1s, w1r, b1, w2, b2):
    grid = BATCH // _BLK
    return pl.pallas_call(
        _mlp_body,
        grid=(grid,),
        in_specs=[
            pl.BlockSpec((_BLK, EMB), lambda i: (i, 0)),
            pl.BlockSpec((_BLK, EMB), lambda i: (i, 0)),
            pl.BlockSpec((EMB, HID), lambda i: (0, 0)),
            pl.BlockSpec((EMB, HID), lambda i: (0, 0)),
            pl.BlockSpec((1, HID), lambda i: (0, 0)),
            pl.BlockSpec((1, HID), lambda i: (0, 0)),
            pl.BlockSpec((1, 1), lambda i: (0, 0)),
        ],
        out_specs=pl.BlockSpec((_BLK,), lambda i: (i,)),
        out_shape=jax.ShapeDtypeStruct((BATCH,), jnp.float32),
    )(s_emb, r_emb, w1s, w1r, b1, w2, b2)


def kernel(sender_idx_batch, receiver_idx_batch, sender_table, receiver_table,
           W1, b1, W2, b2):
    sidx = sender_idx_batch.astype(jnp.int32)
    ridx = receiver_idx_batch.astype(jnp.int32)
    s_emb, r_emb = _gather_call(sidx, ridx, sender_table, receiver_table)
    w1s = W1[:, :EMB].T
    w1r = W1[:, EMB:].T
    b1r = b1.reshape(1, HID)
    w2r = W2.reshape(1, HID)
    b2r = b2.reshape(1, 1)
    return _mlp_call(s_emb, r_emb, w1s, w1r, b1r, w2r, b2r)


# speedup vs baseline: 1.4568x; 1.0082x over previous
import functools

import jax
import jax.numpy as jnp
from jax import lax
from jax.experimental import pallas as pl
from jax.experimental.pallas import tpu as pltpu
from jax.experimental.pallas import tpu_sc as plsc

BATCH = 16384
EMB = 32
HID = 64

_NC = 2
_NS = 16
_NW = _NC * _NS
_BPW = BATCH // _NW
_LAG = 64


def _gather_one_table(idx_h, tab_h, out_h, wid, idx_v, rows_v, sem):
    base = wid * _BPW
    pltpu.sync_copy(idx_h.at[pl.ds(base, _BPW)], idx_v)

    def wait_one():
        pltpu.make_async_copy(
            tab_h.at[pl.ds(0, 1)], rows_v.at[pl.ds(0, 1)], sem).wait()

    def body(g, _):
        vec = idx_v[pl.ds(g * 16, 16)]
        for l in range(16):
            pltpu.async_copy(tab_h.at[pl.ds(vec[l], 1)],
                             rows_v.at[pl.ds(g * 16 + l, 1)], sem)

        @pl.when(g >= _LAG // 16)
        def _():
            for _i in range(16):
                wait_one()
        return 0

    lax.fori_loop(0, _BPW // 16, body, 0)
    for _ in range(_LAG):
        wait_one()
    pltpu.sync_copy(rows_v, out_h.at[pl.ds(base, _BPW)])


def _gather_body(sidx_h, ridx_h, stab_h, rtab_h, sout_h, rout_h,
                 idx_v, rows_v, sem):
    wid = lax.axis_index("s") * _NC + lax.axis_index("c")
    _gather_one_table(sidx_h, stab_h, sout_h, wid, idx_v, rows_v, sem)
    _gather_one_table(ridx_h, rtab_h, rout_h, wid, idx_v, rows_v, sem)


_gather_call = functools.partial(
    pl.kernel,
    out_type=[jax.ShapeDtypeStruct((BATCH, EMB), jnp.float32),
              jax.ShapeDtypeStruct((BATCH, EMB), jnp.float32)],
    mesh=plsc.VectorSubcoreMesh(core_axis_name="c", subcore_axis_name="s"),
    scratch_types=[pltpu.VMEM((_BPW,), jnp.int32),
                   pltpu.VMEM((_BPW, EMB), jnp.float32),
                   pltpu.SemaphoreType.DMA],
    compiler_params=pltpu.CompilerParams(needs_layout_passes=False),
)(_gather_body)


_BLK = 1024


def _mlp_body(s_ref, r_ref, w1s_ref, w1r_ref, b1_ref, w2_ref, b2_ref, out_ref):
    h = jnp.tanh(
        jnp.dot(s_ref[...], w1s_ref[...], preferred_element_type=jnp.float32)
        + jnp.dot(r_ref[...], w1r_ref[...], preferred_element_type=jnp.float32)
        + b1_ref[...])
    logit = jnp.sum(h * w2_ref[...], axis=1) + b2_ref[0, 0]
    out_ref[...] = jax.nn.sigmoid(logit)


def _mlp_call(s_emb, r_emb, w1s, w1r, b1, w2, b2):
    grid = BATCH // _BLK
    return pl.pallas_call(
        _mlp_body,
        grid=(grid,),
        in_specs=[
            pl.BlockSpec((_BLK, EMB), lambda i: (i, 0)),
            pl.BlockSpec((_BLK, EMB), lambda i: (i, 0)),
            pl.BlockSpec((EMB, HID), lambda i: (0, 0)),
            pl.BlockSpec((EMB, HID), lambda i: (0, 0)),
            pl.BlockSpec((1, HID), lambda i: (0, 0)),
            pl.BlockSpec((1, HID), lambda i: (0, 0)),
            pl.BlockSpec((1, 1), lambda i: (0, 0)),
        ],
        out_specs=pl.BlockSpec((_BLK,), lambda i: (i,)),
        out_shape=jax.ShapeDtypeStruct((BATCH,), jnp.float32),
    )(s_emb, r_emb, w1s, w1r, b1, w2, b2)


def kernel(sender_idx_batch, receiver_idx_batch, sender_table, receiver_table,
           W1, b1, W2, b2):
    sidx = sender_idx_batch.astype(jnp.int32)
    ridx = receiver_idx_batch.astype(jnp.int32)
    s_emb, r_emb = _gather_call(sidx, ridx, sender_table, receiver_table)
    w1s = W1[:, :EMB].T
    w1r = W1[:, EMB:].T
    b1r = b1.reshape(1, HID)
    w2r = W2.reshape(1, HID)
    b2r = b2.reshape(1, 1)
    return _mlp_call(s_emb, r_emb, w1s, w1r, b1r, w2r, b2r)
